# baseline (speedup 1.0000x reference)
import jax
import jax.numpy as jnp
from jax.experimental import pallas as pl
from jax.experimental.pallas import tpu as pltpu

_XMINS = jnp.array([1.0728769e-07, 2.0230031, 8.1847715e-08, 136027.72, 3.0, 0.03517608, 3.4028237e+38, 2.0, 0.010238367, 1.396856e-09, 2.6865074e-10, 1.3970158e-09, 0.66699225, 0.66699225, 0.66699225, 0.75000001, 0.61141304, 0.75000001, 0.0, 7.0958774e-08, 0.0], dtype=jnp.float32)
_ALPHAS = jnp.array([1.0868737, 1.0568325, 1.082071, 27.1194, 1.0678725, 1.189603, 443107.0, 1.4543958, 1.1181921, 1.0846596, 1.0769672, 1.085144, 23.798765, 23.798765, 23.922783, 121435080.0, 3.4316596, 121435080.0, 1.08212, 1.082116, 1.08212], dtype=jnp.float32)

_CB = 8


def _body(bfac_ref, rr_ref, coin_ref, x_ref, o_ref):
    b = pl.program_id(0)
    f = bfac_ref[b]
    rr = rr_ref[0]
    cn = coin_ref[0]
    x = x_ref[...]
    ih = jax.lax.broadcasted_iota(jnp.int32, x.shape, 2)
    iw = jax.lax.broadcasted_iota(jnp.int32, x.shape, 3)
    idx = jnp.where(cn == 1, iw, ih)
    o_ref[...] = jnp.where(idx == rr, x * f, x)


def kernel(forward_input):
    p = 0.3
    b, c, h, w = forward_input.shape
    key = jax.random.key(42)
    k1, k2, k3, k4, k5 = jax.random.split(key, 5)
    sampled = jax.random.bernoulli(k1, p, (b,))
    rand_row = jax.random.randint(k2, (), 0, h)
    coin = jax.random.bernoulli(k3, 0.5)
    idx = jax.random.randint(k4, (), 0, _ALPHAS.shape[0])
    r = jax.random.uniform(k5, (), dtype=jnp.float32)
    alpha = _ALPHAS[idx]
    x_min = _XMINS[idx]
    rel = x_min * (1.0 - r) ** (-1.0 / (alpha - 1.0))

    bfac = jnp.where(sampled, rel, jnp.float32(1.0)).astype(jnp.float32)
    rr = rand_row.astype(jnp.int32).reshape(1)
    cn = coin.astype(jnp.int32).reshape(1)

    grid = (b, c // _CB)
    return pl.pallas_call(
        _body,
        grid=grid,
        in_specs=[
            pl.BlockSpec(memory_space=pltpu.SMEM),
            pl.BlockSpec(memory_space=pltpu.SMEM),
            pl.BlockSpec(memory_space=pltpu.SMEM),
            pl.BlockSpec((1, _CB, h, w), lambda i, j: (i, j, 0, 0)),
        ],
        out_specs=pl.BlockSpec((1, _CB, h, w), lambda i, j: (i, j, 0, 0)),
        out_shape=jax.ShapeDtypeStruct((b, c, h, w), jnp.float32),
    )(bfac, rr, cn, forward_input)


# speedup vs baseline: 1.0962x; 1.0962x over previous
import jax
import jax.numpy as jnp
from jax.experimental import pallas as pl
from jax.experimental.pallas import tpu as pltpu

_XMINS = jnp.array([1.0728769e-07, 2.0230031, 8.1847715e-08, 136027.72, 3.0, 0.03517608, 3.4028237e+38, 2.0, 0.010238367, 1.396856e-09, 2.6865074e-10, 1.3970158e-09, 0.66699225, 0.66699225, 0.66699225, 0.75000001, 0.61141304, 0.75000001, 0.0, 7.0958774e-08, 0.0], dtype=jnp.float32)
_ALPHAS = jnp.array([1.0868737, 1.0568325, 1.082071, 27.1194, 1.0678725, 1.189603, 443107.0, 1.4543958, 1.1181921, 1.0846596, 1.0769672, 1.085144, 23.798765, 23.798765, 23.922783, 121435080.0, 3.4316596, 121435080.0, 1.08212, 1.082116, 1.08212], dtype=jnp.float32)

_CB = 16


def _body(bfac_ref, rr_ref, coin_ref, x_ref, o_ref):
    b = pl.program_id(0)
    f = bfac_ref[b]
    rr = rr_ref[0]
    cn = coin_ref[0]
    x = x_ref[...]
    ih = jax.lax.broadcasted_iota(jnp.int32, x.shape, 2)
    iw = jax.lax.broadcasted_iota(jnp.int32, x.shape, 3)
    idx = jnp.where(cn == 1, iw, ih)
    o_ref[...] = jnp.where(idx == rr, x * f, x)


def kernel(forward_input):
    p = 0.3
    b, c, h, w = forward_input.shape
    key = jax.random.key(42)
    k1, k2, k3, k4, k5 = jax.random.split(key, 5)
    sampled = jax.random.bernoulli(k1, p, (b,))
    rand_row = jax.random.randint(k2, (), 0, h)
    coin = jax.random.bernoulli(k3, 0.5)
    idx = jax.random.randint(k4, (), 0, _ALPHAS.shape[0])
    r = jax.random.uniform(k5, (), dtype=jnp.float32)
    alpha = _ALPHAS[idx]
    x_min = _XMINS[idx]
    rel = x_min * (1.0 - r) ** (-1.0 / (alpha - 1.0))

    bfac = jnp.where(sampled, rel, jnp.float32(1.0)).astype(jnp.float32)
    rr = rand_row.astype(jnp.int32).reshape(1)
    cn = coin.astype(jnp.int32).reshape(1)

    grid = (b, c // _CB)
    return pl.pallas_call(
        _body,
        grid=grid,
        in_specs=[
            pl.BlockSpec(memory_space=pltpu.SMEM),
            pl.BlockSpec(memory_space=pltpu.SMEM),
            pl.BlockSpec(memory_space=pltpu.SMEM),
            pl.BlockSpec((1, _CB, h, w), lambda i, j: (i, j, 0, 0)),
        ],
        out_specs=pl.BlockSpec((1, _CB, h, w), lambda i, j: (i, j, 0, 0)),
        out_shape=jax.ShapeDtypeStruct((b, c, h, w), jnp.float32),
    )(bfac, rr, cn, forward_input)


# speedup vs baseline: 1.1112x; 1.0137x over previous
import jax
import jax.numpy as jnp
from jax.experimental import pallas as pl
from jax.experimental.pallas import tpu as pltpu

_XMINS = jnp.array([1.0728769e-07, 2.0230031, 8.1847715e-08, 136027.72, 3.0, 0.03517608, 3.4028237e+38, 2.0, 0.010238367, 1.396856e-09, 2.6865074e-10, 1.3970158e-09, 0.66699225, 0.66699225, 0.66699225, 0.75000001, 0.61141304, 0.75000001, 0.0, 7.0958774e-08, 0.0], dtype=jnp.float32)
_ALPHAS = jnp.array([1.0868737, 1.0568325, 1.082071, 27.1194, 1.0678725, 1.189603, 443107.0, 1.4543958, 1.1181921, 1.0846596, 1.0769672, 1.085144, 23.798765, 23.798765, 23.922783, 121435080.0, 3.4316596, 121435080.0, 1.08212, 1.082116, 1.08212], dtype=jnp.float32)

_CB = 32


def _body(bfac_ref, rr_ref, coin_ref, x_ref, o_ref):
    b = pl.program_id(0)
    f = bfac_ref[b]
    rr = rr_ref[0]
    cn = coin_ref[0]
    x = x_ref[...]
    ih = jax.lax.broadcasted_iota(jnp.int32, x.shape, 2)
    iw = jax.lax.broadcasted_iota(jnp.int32, x.shape, 3)
    idx = jnp.where(cn == 1, iw, ih)
    o_ref[...] = jnp.where(idx == rr, x * f, x)


def kernel(forward_input):
    p = 0.3
    b, c, h, w = forward_input.shape
    key = jax.random.key(42)
    k1, k2, k3, k4, k5 = jax.random.split(key, 5)
    sampled = jax.random.bernoulli(k1, p, (b,))
    rand_row = jax.random.randint(k2, (), 0, h)
    coin = jax.random.bernoulli(k3, 0.5)
    idx = jax.random.randint(k4, (), 0, _ALPHAS.shape[0])
    r = jax.random.uniform(k5, (), dtype=jnp.float32)
    alpha = _ALPHAS[idx]
    x_min = _XMINS[idx]
    rel = x_min * (1.0 - r) ** (-1.0 / (alpha - 1.0))

    bfac = jnp.where(sampled, rel, jnp.float32(1.0)).astype(jnp.float32)
    rr = rand_row.astype(jnp.int32).reshape(1)
    cn = coin.astype(jnp.int32).reshape(1)

    grid = (b, c // _CB)
    return pl.pallas_call(
        _body,
        grid=grid,
        in_specs=[
            pl.BlockSpec(memory_space=pltpu.SMEM),
            pl.BlockSpec(memory_space=pltpu.SMEM),
            pl.BlockSpec(memory_space=pltpu.SMEM),
            pl.BlockSpec((1, _CB, h, w), lambda i, j: (i, j, 0, 0)),
        ],
        out_specs=pl.BlockSpec((1, _CB, h, w), lambda i, j: (i, j, 0, 0)),
        out_shape=jax.ShapeDtypeStruct((b, c, h, w), jnp.float32),
    )(bfac, rr, cn, forward_input)


# speedup vs baseline: 1.1160x; 1.0043x over previous
import jax
import jax.numpy as jnp
from jax.experimental import pallas as pl
from jax.experimental.pallas import tpu as pltpu

_XMINS = jnp.array([1.0728769e-07, 2.0230031, 8.1847715e-08, 136027.72, 3.0, 0.03517608, 3.4028237e+38, 2.0, 0.010238367, 1.396856e-09, 2.6865074e-10, 1.3970158e-09, 0.66699225, 0.66699225, 0.66699225, 0.75000001, 0.61141304, 0.75000001, 0.0, 7.0958774e-08, 0.0], dtype=jnp.float32)
_ALPHAS = jnp.array([1.0868737, 1.0568325, 1.082071, 27.1194, 1.0678725, 1.189603, 443107.0, 1.4543958, 1.1181921, 1.0846596, 1.0769672, 1.085144, 23.798765, 23.798765, 23.922783, 121435080.0, 3.4316596, 121435080.0, 1.08212, 1.082116, 1.08212], dtype=jnp.float32)

_CB = 48


def _body(bfac_ref, rr_ref, coin_ref, x_ref, o_ref):
    b = pl.program_id(0)
    f = bfac_ref[b]
    rr = rr_ref[0]
    cn = coin_ref[0]
    x = x_ref[...]
    ih = jax.lax.broadcasted_iota(jnp.int32, x.shape, 2)
    iw = jax.lax.broadcasted_iota(jnp.int32, x.shape, 3)
    idx = jnp.where(cn == 1, iw, ih)
    o_ref[...] = jnp.where(idx == rr, x * f, x)


def kernel(forward_input):
    p = 0.3
    b, c, h, w = forward_input.shape
    key = jax.random.key(42)
    k1, k2, k3, k4, k5 = jax.random.split(key, 5)
    sampled = jax.random.bernoulli(k1, p, (b,))
    rand_row = jax.random.randint(k2, (), 0, h)
    coin = jax.random.bernoulli(k3, 0.5)
    idx = jax.random.randint(k4, (), 0, _ALPHAS.shape[0])
    r = jax.random.uniform(k5, (), dtype=jnp.float32)
    alpha = _ALPHAS[idx]
    x_min = _XMINS[idx]
    rel = x_min * (1.0 - r) ** (-1.0 / (alpha - 1.0))

    bfac = jnp.where(sampled, rel, jnp.float32(1.0)).astype(jnp.float32)
    rr = rand_row.astype(jnp.int32).reshape(1)
    cn = coin.astype(jnp.int32).reshape(1)

    grid = (b, c // _CB)
    return pl.pallas_call(
        _body,
        grid=grid,
        in_specs=[
            pl.BlockSpec(memory_space=pltpu.SMEM),
            pl.BlockSpec(memory_space=pltpu.SMEM),
            pl.BlockSpec(memory_space=pltpu.SMEM),
            pl.BlockSpec((1, _CB, h, w), lambda i, j: (i, j, 0, 0)),
        ],
        out_specs=pl.BlockSpec((1, _CB, h, w), lambda i, j: (i, j, 0, 0)),
        out_shape=jax.ShapeDtypeStruct((b, c, h, w), jnp.float32),
    )(bfac, rr, cn, forward_input)


# speedup vs baseline: 1.1171x; 1.0010x over previous
import jax
import jax.numpy as jnp
from jax.experimental import pallas as pl
from jax.experimental.pallas import tpu as pltpu

_XMINS = jnp.array([1.0728769e-07, 2.0230031, 8.1847715e-08, 136027.72, 3.0, 0.03517608, 3.4028237e+38, 2.0, 0.010238367, 1.396856e-09, 2.6865074e-10, 1.3970158e-09, 0.66699225, 0.66699225, 0.66699225, 0.75000001, 0.61141304, 0.75000001, 0.0, 7.0958774e-08, 0.0], dtype=jnp.float32)
_ALPHAS = jnp.array([1.0868737, 1.0568325, 1.082071, 27.1194, 1.0678725, 1.189603, 443107.0, 1.4543958, 1.1181921, 1.0846596, 1.0769672, 1.085144, 23.798765, 23.798765, 23.922783, 121435080.0, 3.4316596, 121435080.0, 1.08212, 1.082116, 1.08212], dtype=jnp.float32)

_CB = 48


def _body(bfac_ref, rr_ref, coin_ref, x_ref, o_ref):
    b = pl.program_id(0)
    f = bfac_ref[b]
    rr = rr_ref[0]
    cn = coin_ref[0]
    h, w = x_ref.shape[2], x_ref.shape[3]
    ih = jax.lax.broadcasted_iota(jnp.int32, (h, w), 0)
    iw = jax.lax.broadcasted_iota(jnp.int32, (h, w), 1)
    hit = jnp.where(cn == 1, iw, ih) == rr
    plane = jnp.where(hit, f, jnp.float32(1.0))
    o_ref[...] = x_ref[...] * plane[None, None]


def kernel(forward_input):
    p = 0.3
    b, c, h, w = forward_input.shape
    key = jax.random.key(42)
    k1, k2, k3, k4, k5 = jax.random.split(key, 5)
    sampled = jax.random.bernoulli(k1, p, (b,))
    rand_row = jax.random.randint(k2, (), 0, h)
    coin = jax.random.bernoulli(k3, 0.5)
    idx = jax.random.randint(k4, (), 0, _ALPHAS.shape[0])
    r = jax.random.uniform(k5, (), dtype=jnp.float32)
    alpha = _ALPHAS[idx]
    x_min = _XMINS[idx]
    rel = x_min * (1.0 - r) ** (-1.0 / (alpha - 1.0))

    bfac = jnp.where(sampled, rel, jnp.float32(1.0)).astype(jnp.float32)
    rr = rand_row.astype(jnp.int32).reshape(1)
    cn = coin.astype(jnp.int32).reshape(1)

    grid = (b, c // _CB)
    return pl.pallas_call(
        _body,
        grid=grid,
        in_specs=[
            pl.BlockSpec(memory_space=pltpu.SMEM),
            pl.BlockSpec(memory_space=pltpu.SMEM),
            pl.BlockSpec(memory_space=pltpu.SMEM),
            pl.BlockSpec((1, _CB, h, w), lambda i, j: (i, j, 0, 0)),
        ],
        out_specs=pl.BlockSpec((1, _CB, h, w), lambda i, j: (i, j, 0, 0)),
        out_shape=jax.ShapeDtypeStruct((b, c, h, w), jnp.float32),
    )(bfac, rr, cn, forward_input)
